# Initial kernel scaffold; baseline (speedup 1.0000x reference)
import jax
import jax.numpy as jnp
from jax.experimental import pallas as pl


def kernel(x, target):
    raise NotImplementedError("write your pallas kernel here")


# speedup vs baseline: 6.0917x; 6.0917x over previous
import math

import jax
import jax.numpy as jnp
from jax.experimental import pallas as pl
from jax.experimental.pallas import tpu as pltpu

_V = 100000
_PAD_IDX = 0
_SMOOTH = 0.1
_CONF = 1.0 - _SMOOTH
_EPS = _SMOOTH / (_V - 2)
_ENT = (_V - 1) * _EPS * math.log(_EPS) + _CONF * math.log(_CONF)

_RB = 64


def _loss_kernel(tgt_ref, val_ref, x_ref, out_ref):
    j = pl.program_id(0)
    valid = val_ref[:, :]
    @pl.when(j == 0)
    def _init():
        out_ref[0, 0] = 0.0
    cols = jax.lax.broadcasted_iota(jnp.int32, x_ref.shape, 1)
    hit = cols == tgt_ref[:, :]
    w = jnp.where(hit, valid * jnp.float32(_CONF), valid * jnp.float32(_EPS))
    out_ref[0, 0] += (jnp.float32(_ENT) * jnp.sum(valid)
                      - jnp.sum(w * x_ref[:, :]))


def kernel(x, target):
    B, S, V = x.shape
    R = B * S
    x2 = x.reshape(R, V)
    tgt = target.astype(jnp.int32).reshape(R, 1)
    s_idx = jax.lax.broadcasted_iota(jnp.int32, (B, S), 1).reshape(R, 1)
    valid = ((tgt != _PAD_IDX) & (s_idx != _PAD_IDX)).astype(jnp.float32)
    out = pl.pallas_call(
        _loss_kernel,
        grid=(R // _RB,),
        in_specs=[
            pl.BlockSpec((_RB, 1), lambda j: (j, 0)),
            pl.BlockSpec((_RB, 1), lambda j: (j, 0)),
            pl.BlockSpec((_RB, V), lambda j: (j, 0)),
        ],
        out_specs=pl.BlockSpec((1, 1), lambda j: (0, 0),
                               memory_space=pltpu.SMEM),
        out_shape=jax.ShapeDtypeStruct((1, 1), jnp.float32),
    )(tgt, valid, x2)
    return out[0, 0]
